# baseline (speedup 1.0000x reference)
import functools

import jax
import jax.numpy as jnp
from jax.experimental import pallas as pl
from jax.experimental.pallas import tpu as pltpu
from jax.experimental.pallas import tpu_sc as plsc

B, N_J, N_M, N_OPE = 64, 20, 20, 400
IN_DIM, HID, N_LAYERS, EMBD = 3, 128, 3, 2

def _dg(a, b):
    return jax.lax.dot_general(a, b, (((1,), (1,)), ((), ())),
                               preferred_element_type=jnp.float32)


def _dg_hi(a, b):
    return jax.lax.dot_general(a, b, (((1,), (1,)), ((), ())),
                               precision=jax.lax.Precision.HIGHEST,
                               preferred_element_type=jnp.float32)


def _dot(a, b):
    return jax.lax.dot_general(a, b, (((1,), (0,)), ((), ())),
                               preferred_element_type=jnp.float32)


def _assign_body(pref, fc1_w, fc1_b, fc2_w, fc2_b, fc3_w, fc3_b, m_o):
    m = _dg(pref[...], fc1_w[...]) + fc1_b[...]
    m = _dg(m, fc2_w[...]) + fc2_b[...]
    m_o[...] = _dg(m, fc3_w[...]) + fc3_b[...]


def _assign(pref, p):
    args = [pref.reshape(1, 3).astype(jnp.float32),
            p["fc1_w"], p["fc1_b"].reshape(1, 256),
            p["fc2_w"], p["fc2_b"].reshape(1, 256),
            p["fc3_w"], p["fc3_b"].reshape(1, EMBD * 6)]
    return pl.pallas_call(
        _assign_body,
        out_shape=jax.ShapeDtypeStruct((1, EMBD * 6), jnp.float32))(*args)


def _sc_gather_body(idx_hbm, tbl_hbm, out_hbm, idx_v, rows_v, sem):
    c = jax.lax.axis_index("c")
    s = jax.lax.axis_index("s")

    @pl.when(jnp.logical_and(c == 0, s == 0))
    def _():
        pltpu.sync_copy(idx_hbm, idx_v)
        pltpu.async_copy(tbl_hbm.at[idx_v], rows_v, sem).wait()
        pltpu.sync_copy(rows_v, out_hbm)


_SC_W = 2 * N_M

_sc_gather = pl.kernel(
    _sc_gather_body,
    out_type=jax.ShapeDtypeStruct((B, _SC_W), jnp.float32),
    mesh=plsc.VectorSubcoreMesh(core_axis_name="c", subcore_axis_name="s"),
    compiler_params=pltpu.CompilerParams(use_tc_tiling_on_sc=False),
    scratch_types=[pltpu.VMEM((B,), jnp.int32),
                   pltpu.VMEM((B, _SC_W), jnp.float32),
                   pltpu.SemaphoreType.DMA],
)


_IPP = 4


def _main_body(x_r, adj_r, gp_r, cand_r, maskf_r, eps_r, m_r,
               w10, b10, w20, b20, w11, b11, w21, b21, w12, b12, w22, b22,
               w1a, w1b, l1b, bw1a, bw1b, b1b,
               w2a, w2b, l2b, bw2a, bw2b, b2b,
               w3a, w3b, l3b, b3w, b3b,
               ints_o, hp_o, af_o, h_scr, cc_scr):
    def _r(s):
        return s.astype(jnp.bfloat16).astype(jnp.float32)

    mm = [_r(m_r[0, i]) for i in range(12)]
    d1 = (mm[0] * w1a[...] + mm[1] * w1b[...]) + l1b[...]
    db1 = (mm[2] * bw1a[...] + mm[3] * bw1b[...]) + b1b[...]
    d2 = (mm[4] * w2a[...] + mm[5] * w2b[...]) + l2b[...]
    db2 = (mm[6] * bw2a[...] + mm[7] * bw2b[...]) + b2b[...]
    d3 = (mm[8] * w3a[...] + mm[9] * w3b[...]) + l3b[...]
    db3 = (mm[10] * b3w[0, 0] + mm[11] * b3w[0, 1]) + b3b[0, 0]

    gnn = ((w10, b10, w20, b20), (w11, b11, w21, b21), (w12, b12, w22, b22))
    rng = range(_IPP)
    hs = [x_r[i] for i in rng]
    for l in range(N_LAYERS):
        w1, b1, w2, b2 = gnn[l]
        pooled = [_dot(adj_r[i], hs[i]) for i in rng]
        h2 = [(1.0 + eps_r[0, l]) * hs[i] + pooled[i] for i in rng]
        h2 = [jax.nn.relu(_dg(h2[i], w1[...]) + b1[...]) for i in rng]
        hs = [jax.nn.relu(_dg(h2[i], w2[...]) + b2[...]) for i in rng]
    hps = [_dot(gp_r[i], hs[i]) for i in rng]

    for i in rng:
        h_scr[i] = hs[i]
        cc_scr[i, :, HID:] = jnp.broadcast_to(hps[i], (N_J, HID))
        for j in range(N_J):
            cc_scr[i, j:j + 1, :HID] = h_scr[i, pl.ds(cand_r[i, 0, j], 1), :]
    t = [jnp.tanh(_dg(cc_scr[i], d1) + db1) for i in rng]
    t = [jnp.tanh(_dg(t[i], d2) + db2) for i in rng]
    scores = [(_dg(d3, t[i]) + db3) * 10.0 for i in rng]
    lane = jax.lax.broadcasted_iota(jnp.int32, (1, N_J), 1)
    lane8 = jax.lax.broadcasted_iota(jnp.int32, (1, 8), 1)
    for i in rng:
        masked = jnp.where(maskf_r[i] > 0.5, float("-inf"), scores[i])
        mx = jnp.max(masked, axis=1, keepdims=True)
        index = jnp.min(jnp.where(masked == mx, lane, 2 ** 30))
        action = jnp.sum(jnp.where(lane == index, cand_r[i], 0))

        af_o[i] = h_scr[i, pl.ds(action, 1), :]
        hp_o[i] = hps[i]
        row = N_OPE * (_IPP * pl.program_id(0) + i) + action
        ints_o[i] = jnp.where(lane8 == 0, action,
                              jnp.where(lane8 == 1, index,
                                        jnp.where(lane8 == 2, row, 0)))


def kernel(x, graph_pool, padded_nei, adj, candidate, mask, mask_mch, dur,
           a_index, old_action, mch_pool, pref, params):
    del padded_nei, a_index, old_action, mch_pool
    p = params
    m = _assign(pref, p)

    gp3 = graph_pool.reshape(B, 1, N_OPE)
    cand3 = candidate.astype(jnp.int32).reshape(B, 1, N_J)
    maskf3 = mask.astype(jnp.float32).reshape(B, 1, N_J)
    mmf = mask_mch.astype(jnp.float32)
    eps2 = p["eps"].reshape(1, N_LAYERS)

    def fixed(shape):
        n = len(shape)
        return pl.BlockSpec(shape, lambda b, _n=n: (0,) * _n)

    in_specs = [
        pl.BlockSpec((_IPP, N_OPE, IN_DIM), lambda b: (b, 0, 0)),
        pl.BlockSpec((_IPP, N_OPE, N_OPE), lambda b: (b, 0, 0)),
        pl.BlockSpec((_IPP, 1, N_OPE), lambda b: (b, 0, 0)),
        pl.BlockSpec((_IPP, 1, N_J), lambda b: (b, 0, 0)),
        pl.BlockSpec((_IPP, 1, N_J), lambda b: (b, 0, 0)),
        fixed((1, N_LAYERS)),
        fixed((1, EMBD * 6)),
    ]
    weight_args = [m]
    for l in range(N_LAYERS):
        lay = p["gnn"][l]
        weight_args += [lay["w1"], lay["b1"].reshape(1, HID),
                        lay["w2"], lay["b2"].reshape(1, HID)]
        in_specs += [fixed(tuple(lay["w1"].shape)), fixed((1, HID)),
                     fixed((HID, HID)), fixed((1, HID))]
    def rd(a):
        return a.astype(jnp.bfloat16).astype(jnp.float32)

    weight_args += [
        rd(p["lin1_w"][:, 0]).reshape(HID, 256), rd(p["lin1_w"][:, 1]).reshape(HID, 256),
        p["lin1_b"].reshape(HID, 256),
        rd(p["bias1_w"][:, 0]).reshape(1, HID), rd(p["bias1_w"][:, 1]).reshape(1, HID),
        p["bias1_b"].reshape(1, HID),
        rd(p["lin2_w"][:, 0]).reshape(HID, HID), rd(p["lin2_w"][:, 1]).reshape(HID, HID),
        p["lin2_b"].reshape(HID, HID),
        rd(p["bias2_w"][:, 0]).reshape(1, HID), rd(p["bias2_w"][:, 1]).reshape(1, HID),
        p["bias2_b"].reshape(1, HID),
        rd(p["lin3_w"][:, 0]).reshape(1, HID), rd(p["lin3_w"][:, 1]).reshape(1, HID),
        p["lin3_b"].reshape(1, HID),
        rd(p["bias3_w"]), p["bias3_b"].reshape(1, 1),
    ]
    in_specs += [fixed((HID, 256)), fixed((HID, 256)), fixed((HID, 256)),
                 fixed((1, HID)), fixed((1, HID)), fixed((1, HID)),
                 fixed((HID, HID)), fixed((HID, HID)), fixed((HID, HID)),
                 fixed((1, HID)), fixed((1, HID)), fixed((1, HID)),
                 fixed((1, HID)), fixed((1, HID)), fixed((1, HID)),
                 fixed((1, EMBD)), fixed((1, 1))]

    out_shapes = [
        jax.ShapeDtypeStruct((B, 1, 8), jnp.int32),
        jax.ShapeDtypeStruct((B, 1, HID), jnp.float32),
        jax.ShapeDtypeStruct((B, 1, HID), jnp.float32),
    ]
    out_specs = [
        pl.BlockSpec((_IPP, 1, 8), lambda b: (b, 0, 0)),
        pl.BlockSpec((_IPP, 1, HID), lambda b: (b, 0, 0)),
        pl.BlockSpec((_IPP, 1, HID), lambda b: (b, 0, 0)),
    ]
    ints, hp, af = pl.pallas_call(
        _main_body,
        grid=(B // _IPP,),
        in_specs=in_specs,
        out_specs=out_specs,
        out_shape=out_shapes,
        scratch_shapes=[pltpu.VMEM((_IPP, N_OPE, HID), jnp.float32),
                        pltpu.VMEM((_IPP, N_J, 2 * HID), jnp.float32)],
    )(x, adj, gp3, cand3, maskf3, eps2, *weight_args)

    action = ints[:, 0, 0]
    index = ints[:, 0, 1]
    rows = ints[:, 0, 2]
    tbl = jnp.concatenate([dur, mmf], axis=2).reshape(B * N_OPE, _SC_W)
    g = _sc_gather(rows, tbl)
    log_a = jnp.zeros((), jnp.float32)
    return (action, index, log_a, g[:, :N_M], af.reshape(B, HID),
            (g[:, N_M:] != 0).reshape(B, 1, N_M), hp.reshape(B, HID))


# speedup vs baseline: 1.0495x; 1.0495x over previous
import functools

import jax
import jax.numpy as jnp
from jax.experimental import pallas as pl
from jax.experimental.pallas import tpu as pltpu
from jax.experimental.pallas import tpu_sc as plsc

B, N_J, N_M, N_OPE = 64, 20, 20, 400
IN_DIM, HID, N_LAYERS, EMBD = 3, 128, 3, 2

def _dg(a, b):
    return jax.lax.dot_general(a, b, (((1,), (1,)), ((), ())),
                               preferred_element_type=jnp.float32)


def _dg_hi(a, b):
    return jax.lax.dot_general(a, b, (((1,), (1,)), ((), ())),
                               precision=jax.lax.Precision.HIGHEST,
                               preferred_element_type=jnp.float32)


def _dot(a, b):
    return jax.lax.dot_general(a, b, (((1,), (0,)), ((), ())),
                               preferred_element_type=jnp.float32)


def _assign_body(pref, fc1_w, fc1_b, fc2_w, fc2_b, fc3_w, fc3_b, m_o):
    m = _dg(pref[...], fc1_w[...]) + fc1_b[...]
    m = _dg(m, fc2_w[...]) + fc2_b[...]
    m_o[...] = _dg(m, fc3_w[...]) + fc3_b[...]


def _assign(pref, p):
    args = [pref.reshape(1, 3).astype(jnp.float32),
            p["fc1_w"], p["fc1_b"].reshape(1, 256),
            p["fc2_w"], p["fc2_b"].reshape(1, 256),
            p["fc3_w"], p["fc3_b"].reshape(1, EMBD * 6)]
    return pl.pallas_call(
        _assign_body,
        out_shape=jax.ShapeDtypeStruct((1, EMBD * 6), jnp.float32))(*args)


def _sc_gather_body(idx_hbm, tbl_hbm, out_hbm, idx_v, rows_v, sem):
    c = jax.lax.axis_index("c")
    s = jax.lax.axis_index("s")

    @pl.when(jnp.logical_and(c == 0, s == 0))
    def _():
        pltpu.sync_copy(idx_hbm, idx_v)
        pltpu.async_copy(tbl_hbm.at[idx_v], rows_v, sem).wait()
        pltpu.sync_copy(rows_v, out_hbm)


_SC_W = 2 * N_M

_sc_gather = pl.kernel(
    _sc_gather_body,
    out_type=jax.ShapeDtypeStruct((B, _SC_W), jnp.float32),
    mesh=plsc.VectorSubcoreMesh(core_axis_name="c", subcore_axis_name="s"),
    compiler_params=pltpu.CompilerParams(use_tc_tiling_on_sc=False),
    scratch_types=[pltpu.VMEM((B,), jnp.int32),
                   pltpu.VMEM((B, _SC_W), jnp.float32),
                   pltpu.SemaphoreType.DMA],
)


_IPP = 8


def _main_body(x_r, adj_r, gp_r, cand_r, maskf_r, eps_r, m_r,
               w10, b10, w20, b20, w11, b11, w21, b21, w12, b12, w22, b22,
               w1a, w1b, l1b, bw1a, bw1b, b1b,
               w2a, w2b, l2b, bw2a, bw2b, b2b,
               w3a, w3b, l3b, b3w, b3b,
               ints_o, hp_o, af_o, h_scr, cc_scr):
    def _r(s):
        return s.astype(jnp.bfloat16).astype(jnp.float32)

    mm = [_r(m_r[0, i]) for i in range(12)]
    d1 = (mm[0] * w1a[...] + mm[1] * w1b[...]) + l1b[...]
    db1 = (mm[2] * bw1a[...] + mm[3] * bw1b[...]) + b1b[...]
    d2 = (mm[4] * w2a[...] + mm[5] * w2b[...]) + l2b[...]
    db2 = (mm[6] * bw2a[...] + mm[7] * bw2b[...]) + b2b[...]
    d3 = (mm[8] * w3a[...] + mm[9] * w3b[...]) + l3b[...]
    db3 = (mm[10] * b3w[0, 0] + mm[11] * b3w[0, 1]) + b3b[0, 0]

    gnn = ((w10, b10, w20, b20), (w11, b11, w21, b21), (w12, b12, w22, b22))
    rng = range(_IPP)
    hs = [x_r[i] for i in rng]
    for l in range(N_LAYERS):
        w1, b1, w2, b2 = gnn[l]
        pooled = [_dot(adj_r[i], hs[i]) for i in rng]
        h2 = [(1.0 + eps_r[0, l]) * hs[i] + pooled[i] for i in rng]
        h2 = [jax.nn.relu(_dg(h2[i], w1[...]) + b1[...]) for i in rng]
        hs = [jax.nn.relu(_dg(h2[i], w2[...]) + b2[...]) for i in rng]
    hps = [_dot(gp_r[i], hs[i]) for i in rng]

    for i in rng:
        h_scr[i] = hs[i]
        cc_scr[i, :, HID:] = jnp.broadcast_to(hps[i], (N_J, HID))
        for j in range(N_J):
            cc_scr[i, j:j + 1, :HID] = h_scr[i, pl.ds(cand_r[i, 0, j], 1), :]
    t = [jnp.tanh(_dg(cc_scr[i], d1) + db1) for i in rng]
    t = [jnp.tanh(_dg(t[i], d2) + db2) for i in rng]
    scores = [(_dg(d3, t[i]) + db3) * 10.0 for i in rng]
    lane = jax.lax.broadcasted_iota(jnp.int32, (1, N_J), 1)
    lane8 = jax.lax.broadcasted_iota(jnp.int32, (1, 8), 1)
    for i in rng:
        masked = jnp.where(maskf_r[i] > 0.5, float("-inf"), scores[i])
        mx = jnp.max(masked, axis=1, keepdims=True)
        index = jnp.min(jnp.where(masked == mx, lane, 2 ** 30))
        action = jnp.sum(jnp.where(lane == index, cand_r[i], 0))

        af_o[i] = h_scr[i, pl.ds(action, 1), :]
        hp_o[i] = hps[i]
        row = N_OPE * (_IPP * pl.program_id(0) + i) + action
        ints_o[i] = jnp.where(lane8 == 0, action,
                              jnp.where(lane8 == 1, index,
                                        jnp.where(lane8 == 2, row, 0)))


def kernel(x, graph_pool, padded_nei, adj, candidate, mask, mask_mch, dur,
           a_index, old_action, mch_pool, pref, params):
    del padded_nei, a_index, old_action, mch_pool
    p = params
    m = _assign(pref, p)

    gp3 = graph_pool.reshape(B, 1, N_OPE)
    cand3 = candidate.astype(jnp.int32).reshape(B, 1, N_J)
    maskf3 = mask.astype(jnp.float32).reshape(B, 1, N_J)
    mmf = mask_mch.astype(jnp.float32)
    eps2 = p["eps"].reshape(1, N_LAYERS)

    def fixed(shape):
        n = len(shape)
        return pl.BlockSpec(shape, lambda b, _n=n: (0,) * _n)

    in_specs = [
        pl.BlockSpec((_IPP, N_OPE, IN_DIM), lambda b: (b, 0, 0)),
        pl.BlockSpec((_IPP, N_OPE, N_OPE), lambda b: (b, 0, 0)),
        pl.BlockSpec((_IPP, 1, N_OPE), lambda b: (b, 0, 0)),
        pl.BlockSpec((_IPP, 1, N_J), lambda b: (b, 0, 0)),
        pl.BlockSpec((_IPP, 1, N_J), lambda b: (b, 0, 0)),
        fixed((1, N_LAYERS)),
        fixed((1, EMBD * 6)),
    ]
    weight_args = [m]
    for l in range(N_LAYERS):
        lay = p["gnn"][l]
        weight_args += [lay["w1"], lay["b1"].reshape(1, HID),
                        lay["w2"], lay["b2"].reshape(1, HID)]
        in_specs += [fixed(tuple(lay["w1"].shape)), fixed((1, HID)),
                     fixed((HID, HID)), fixed((1, HID))]
    def rd(a):
        return a.astype(jnp.bfloat16).astype(jnp.float32)

    weight_args += [
        rd(p["lin1_w"][:, 0]).reshape(HID, 256), rd(p["lin1_w"][:, 1]).reshape(HID, 256),
        p["lin1_b"].reshape(HID, 256),
        rd(p["bias1_w"][:, 0]).reshape(1, HID), rd(p["bias1_w"][:, 1]).reshape(1, HID),
        p["bias1_b"].reshape(1, HID),
        rd(p["lin2_w"][:, 0]).reshape(HID, HID), rd(p["lin2_w"][:, 1]).reshape(HID, HID),
        p["lin2_b"].reshape(HID, HID),
        rd(p["bias2_w"][:, 0]).reshape(1, HID), rd(p["bias2_w"][:, 1]).reshape(1, HID),
        p["bias2_b"].reshape(1, HID),
        rd(p["lin3_w"][:, 0]).reshape(1, HID), rd(p["lin3_w"][:, 1]).reshape(1, HID),
        p["lin3_b"].reshape(1, HID),
        rd(p["bias3_w"]), p["bias3_b"].reshape(1, 1),
    ]
    in_specs += [fixed((HID, 256)), fixed((HID, 256)), fixed((HID, 256)),
                 fixed((1, HID)), fixed((1, HID)), fixed((1, HID)),
                 fixed((HID, HID)), fixed((HID, HID)), fixed((HID, HID)),
                 fixed((1, HID)), fixed((1, HID)), fixed((1, HID)),
                 fixed((1, HID)), fixed((1, HID)), fixed((1, HID)),
                 fixed((1, EMBD)), fixed((1, 1))]

    out_shapes = [
        jax.ShapeDtypeStruct((B, 1, 8), jnp.int32),
        jax.ShapeDtypeStruct((B, 1, HID), jnp.float32),
        jax.ShapeDtypeStruct((B, 1, HID), jnp.float32),
    ]
    out_specs = [
        pl.BlockSpec((_IPP, 1, 8), lambda b: (b, 0, 0)),
        pl.BlockSpec((_IPP, 1, HID), lambda b: (b, 0, 0)),
        pl.BlockSpec((_IPP, 1, HID), lambda b: (b, 0, 0)),
    ]
    ints, hp, af = pl.pallas_call(
        _main_body,
        grid=(B // _IPP,),
        in_specs=in_specs,
        out_specs=out_specs,
        out_shape=out_shapes,
        scratch_shapes=[pltpu.VMEM((_IPP, N_OPE, HID), jnp.float32),
                        pltpu.VMEM((_IPP, N_J, 2 * HID), jnp.float32)],
    )(x, adj, gp3, cand3, maskf3, eps2, *weight_args)

    action = ints[:, 0, 0]
    index = ints[:, 0, 1]
    rows = ints[:, 0, 2]
    tbl = jnp.concatenate([dur, mmf], axis=2).reshape(B * N_OPE, _SC_W)
    g = _sc_gather(rows, tbl)
    log_a = jnp.zeros((), jnp.float32)
    return (action, index, log_a, g[:, :N_M], af.reshape(B, HID),
            (g[:, N_M:] != 0).reshape(B, 1, N_M), hp.reshape(B, HID))
